# baseline (speedup 1.0000x reference)
import functools

import jax
import jax.numpy as jnp
from jax import lax
from jax.experimental import pallas as pl
from jax.experimental.pallas import tpu as pltpu
from jax.experimental.pallas import tpu_sc as plsc

VOC_SIZE = 1000000
EMB_DIM = 32
BATCH = 16384
HIST_LEN = 200
HALF = 100
HALF_PAD = 128
HALF_G = HALF_PAD
HIST_G = 2 * HALF_G
NHALF = BATCH * 2

_info = plsc.get_sparse_core_info()
NC = _info.num_cores
NS = _info.num_subcores
NW = NC * NS
ROWS_PER_W = BATCH // NW
HALVES_PER_W = 2 * ROWS_PER_W
IDX_CHUNK = 256
NPHASE = HALVES_PER_W // IDX_CHUNK
NBUF = 8
NGROUP = IDX_CHUNK // NBUF


def _count_zeros(idx_ref, r):
    lane = lax.iota(jnp.int32, 16)
    one = jnp.ones((16,), jnp.int32)
    nil = jnp.zeros((16,), jnp.int32)
    cnt = nil
    for o in range(0, HALF_G, 16):
        v = idx_ref[r, pl.ds(o, 16)]
        cnt = cnt + jnp.where(v == 0, one, nil)
    return cnt


def _hsum16(vec, scratch_ref):
    lane = lax.iota(jnp.int32, 16)
    for sh in (8, 4, 2, 1):
        scratch_ref[...] = vec
        vec = vec + plsc.load_gather(scratch_ref, [lane ^ sh])
    return vec


def _body(x2_hbm, w_hbm, out_hbm, idx_v, bufs, out_v, w0_v, hs_v, sems):
    wid = lax.axis_index("s") * NC + lax.axis_index("c")
    base_h = wid * HALVES_PER_W

    pltpu.sync_copy(w_hbm.at[pl.ds(0, 8)], w0_v)
    w0a = w0_v[0, pl.ds(0, 16)]
    w0b = w0_v[0, pl.ds(16, 16)]

    zero = jnp.zeros((16,), jnp.float32)

    def fire(h, b):
        pltpu.async_copy(
            w_hbm.at[idx_v.at[h]], bufs[b], sems[b]
        )

    def drain(h, b):
        pltpu.make_async_copy(
            w_hbm.at[idx_v.at[h]], bufs[b], sems[b]
        ).wait()

    def phase(p, pcarry):
        pltpu.sync_copy(
            x2_hbm.at[pl.ds(base_h + p * IDX_CHUNK, IDX_CHUNK)], idx_v
        )
        for b in range(NBUF):
            fire(b, b)

        def group(g, carry):
            h0 = g * NBUF
            more = g < NGROUP - 1
            for pairb in range(NBUF // 2):
                acc0 = zero
                acc1 = zero
                nz = None
                for b in (2 * pairb, 2 * pairb + 1):
                    h = h0 + b
                    drain(h, b)
                    rv = bufs[b]
                    for i in range(HALF_G):
                        acc0 = acc0 + rv[i, pl.ds(0, 16)]
                        acc1 = acc1 + rv[i, pl.ds(16, 16)]
                    zc = _count_zeros(idx_v, h)
                    nz = zc if nz is None else nz + zc

                    @pl.when(more)
                    def _(h=h, b=b):
                        fire(h + NBUF, b)

                nz = _hsum16(nz, hs_v)
                nzf = nz.astype(jnp.float32)
                cntf = (HIST_G - nz).astype(jnp.float32)
                orow = p * (IDX_CHUNK // 2) + (h0 // 2) + pairb
                out_v[orow, pl.ds(0, 16)] = (acc0 - nzf * w0a) / cntf
                out_v[orow, pl.ds(16, 16)] = (acc1 - nzf * w0b) / cntf
            return carry

        lax.fori_loop(0, NGROUP, group, 0)
        return pcarry

    lax.fori_loop(0, NPHASE, phase, 0)

    pltpu.sync_copy(out_v, out_hbm.at[pl.ds(wid * ROWS_PER_W, ROWS_PER_W)])


@functools.partial(jax.jit, donate_argnums=())
def kernel(X, W):
    x2 = X.astype(jnp.int32).reshape(NHALF, HALF)
    x2 = jnp.pad(x2, ((0, 0), (0, HALF_PAD - HALF)))
    mesh = plsc.VectorSubcoreMesh(core_axis_name="c", subcore_axis_name="s")
    k = pl.kernel(
        _body,
        mesh=mesh,
        out_type=jax.ShapeDtypeStruct((BATCH, EMB_DIM), jnp.float32),
        scratch_types=[
            pltpu.VMEM((IDX_CHUNK, HALF_PAD), jnp.int32),
            [pltpu.VMEM((HALF_G, EMB_DIM), jnp.float32) for _ in range(NBUF)],
            pltpu.VMEM((ROWS_PER_W, EMB_DIM), jnp.float32),
            pltpu.VMEM((8, EMB_DIM), jnp.float32),
            pltpu.VMEM((16,), jnp.int32),
            [pltpu.SemaphoreType.DMA for _ in range(NBUF)],
        ],
        compiler_params=pltpu.CompilerParams(
            needs_layout_passes=False, use_tc_tiling_on_sc=False
        ),
    )
    return k(x2, W)


# speedup vs baseline: 7.5127x; 7.5127x over previous
import functools

import jax
import jax.numpy as jnp
from jax import lax
from jax.experimental import pallas as pl
from jax.experimental.pallas import tpu as pltpu
from jax.experimental.pallas import tpu_sc as plsc

VOC_SIZE = 1000000
EMB_DIM = 32
BATCH = 16384
HIST_LEN = 200
RW = 128
XROWS = BATCH * HIST_LEN // RW

_info = plsc.get_sparse_core_info()
NC = _info.num_cores
NS = _info.num_subcores
NW = NC * NS
ROWS_PER_W = BATCH // NW
XR_PER_W = XROWS // NW
PHASE_XR = 200
NPHASE = XR_PER_W // PHASE_XR
PHASE_BROWS = PHASE_XR * RW // HIST_LEN
NBUF = 8
NGROUP = PHASE_XR // NBUF


def _hsum16(vec, scratch_ref):
    lane = lax.iota(jnp.int32, 16)
    for sh in (8, 4, 2, 1):
        scratch_ref[...] = vec
        vec = vec + plsc.load_gather(scratch_ref, [lane ^ sh])
    return vec


def _body(x_hbm, w_hbm, out_hbm, idx_v, bufs, out_v, w0_v, hs_v, sems):
    wid = lax.axis_index("s") * NC + lax.axis_index("c")
    xr_base = wid * XR_PER_W

    pltpu.sync_copy(w_hbm.at[pl.ds(0, 8)], w0_v)
    w0a = w0_v[0, pl.ds(0, 16)]
    w0b = w0_v[0, pl.ds(16, 16)]

    zero = jnp.zeros((16,), jnp.float32)
    izero = jnp.zeros((16,), jnp.int32)
    ione = jnp.ones((16,), jnp.int32)
    lane = lax.iota(jnp.int32, 16)

    def fire(r, b):
        pltpu.async_copy(w_hbm.at[idx_v.at[r]], bufs[b], sems[b])

    def drain(r, b):
        pltpu.make_async_copy(w_hbm.at[idx_v.at[r]], bufs[b], sems[b]).wait()

    def phase(p, pcarry):
        pltpu.sync_copy(
            x_hbm.at[pl.ds(xr_base + p * PHASE_XR, PHASE_XR)], idx_v
        )
        for b in range(NBUF):
            fire(b, b)

        def group(g, carry):
            acc0, acc1, cnt, j, bpos = carry
            more = g < NGROUP - 1
            for b in range(NBUF):
                r = g * NBUF + b
                fs = (g * NBUF + b) * RW
                sp = bpos - fs
                drain(r, b)
                rv = bufs[b]
                t0 = zero
                t1 = zero
                p0 = zero
                p1 = zero
                for gi in range(RW // 8):
                    s0 = zero
                    s1 = zero
                    for s in range(8 * gi, 8 * gi + 8):
                        s0 = s0 + rv[s, pl.ds(0, 16)]
                        s1 = s1 + rv[s, pl.ds(16, 16)]
                    t0 = t0 + s0
                    t1 = t1 + s1
                    pm = jnp.where(8 * gi < sp, 1.0, 0.0)
                    p0 = p0 + pm * s0
                    p1 = p1 + pm * s1
                ct = izero
                cp = izero
                spv = jnp.full((16,), sp, jnp.int32)
                for o in range(0, RW, 16):
                    v = idx_v[r, pl.ds(o, 16)]
                    z = v == 0
                    ct = ct + jnp.where(z, ione, izero)
                    cp = cp + jnp.where(
                        jnp.logical_and(z, (o + lane) < spv), ione, izero
                    )

                @pl.when(more)
                def _(r=r, b=b):
                    fire(r + NBUF, b)

                fin0 = acc0 + p0
                fin1 = acc1 + p1
                finc = cnt + cp
                has_b = sp <= RW

                @pl.when(has_b)
                def _(fin0=fin0, fin1=fin1, finc=finc, j=j):
                    nz = _hsum16(finc, hs_v)
                    nzf = nz.astype(jnp.float32)
                    cntf = (HIST_LEN - nz).astype(jnp.float32)
                    out_v[j, pl.ds(0, 16)] = (fin0 - nzf * w0a) / cntf
                    out_v[j, pl.ds(16, 16)] = (fin1 - nzf * w0b) / cntf

                hbf = jnp.where(has_b, 1.0, 0.0)
                hbi = jnp.where(has_b, 1, 0)
                acc0 = acc0 + t0 - hbf * fin0
                acc1 = acc1 + t1 - hbf * fin1
                cnt = cnt + ct - hbi * finc
                j = j + hbi
                bpos = bpos + HIST_LEN * hbi
            return (acc0, acc1, cnt, j, bpos)

        j0 = p * PHASE_BROWS
        lax.fori_loop(
            0, NGROUP, group, (zero, zero, izero, j0, jnp.int32(HIST_LEN))
        )
        return pcarry

    lax.fori_loop(0, NPHASE, phase, 0)

    pltpu.sync_copy(out_v, out_hbm.at[pl.ds(wid * ROWS_PER_W, ROWS_PER_W)])


@functools.partial(jax.jit, donate_argnums=())
def kernel(X, W):
    xf = X.astype(jnp.int32).reshape(XROWS, RW)
    mesh = plsc.VectorSubcoreMesh(core_axis_name="c", subcore_axis_name="s")
    k = pl.kernel(
        _body,
        mesh=mesh,
        out_type=jax.ShapeDtypeStruct((BATCH, EMB_DIM), jnp.float32),
        scratch_types=[
            pltpu.VMEM((PHASE_XR, RW), jnp.int32),
            [pltpu.VMEM((RW, EMB_DIM), jnp.float32) for _ in range(NBUF)],
            pltpu.VMEM((ROWS_PER_W, EMB_DIM), jnp.float32),
            pltpu.VMEM((8, EMB_DIM), jnp.float32),
            pltpu.VMEM((16,), jnp.int32),
            [pltpu.SemaphoreType.DMA for _ in range(NBUF)],
        ],
        compiler_params=pltpu.CompilerParams(
            needs_layout_passes=False, use_tc_tiling_on_sc=False
        ),
    )
    return k(xf, W)


# speedup vs baseline: 8.1167x; 1.0804x over previous
import functools

import jax
import jax.numpy as jnp
from jax import lax
from jax.experimental import pallas as pl
from jax.experimental.pallas import tpu as pltpu
from jax.experimental.pallas import tpu_sc as plsc

VOC_SIZE = 1000000
EMB_DIM = 32
BATCH = 16384
HIST_LEN = 200
RW = 128
XROWS = BATCH * HIST_LEN // RW

_info = plsc.get_sparse_core_info()
NC = _info.num_cores
NS = _info.num_subcores
NW = NC * NS
ROWS_PER_W = BATCH // NW
XR_PER_W = XROWS // NW
PHASE_XR = 200
NPHASE = XR_PER_W // PHASE_XR
PHASE_BROWS = PHASE_XR * RW // HIST_LEN
NBUF = 4
NGROUP = PHASE_XR // NBUF


def _hsum16(vec, scratch_ref):
    lane = lax.iota(jnp.int32, 16)
    for sh in (8, 4, 2, 1):
        scratch_ref[...] = vec
        vec = vec + plsc.load_gather(scratch_ref, [lane ^ sh])
    return vec


def _body(x_hbm, w_hbm, out_hbm, idx_v, bufs, out_v, w0_v, hs_v, sems):
    wid = lax.axis_index("s") * NC + lax.axis_index("c")
    xr_base = wid * XR_PER_W

    pltpu.sync_copy(w_hbm.at[pl.ds(0, 8)], w0_v)
    w0a = w0_v[0, pl.ds(0, 16)]
    w0b = w0_v[0, pl.ds(16, 16)]

    zero = jnp.zeros((16,), jnp.float32)
    izero = jnp.zeros((16,), jnp.int32)
    ione = jnp.ones((16,), jnp.int32)
    lane = lax.iota(jnp.int32, 16)

    def fire(r, b):
        pltpu.async_copy(w_hbm.at[idx_v.at[r]], bufs[b], sems[b])

    def drain(r, b):
        pltpu.make_async_copy(w_hbm.at[idx_v.at[r]], bufs[b], sems[b]).wait()

    def phase(p, pcarry):
        pltpu.sync_copy(
            x_hbm.at[pl.ds(xr_base + p * PHASE_XR, PHASE_XR)], idx_v
        )
        for b in range(NBUF):
            fire(b, b)

        def group(g, carry):
            acc0, acc1, cnt, j, bpos = carry
            more = g < NGROUP - 1
            for b in range(NBUF):
                r = g * NBUF + b
                fs = (g * NBUF + b) * RW
                sp = bpos - fs
                drain(r, b)
                rv = bufs[b]
                t0 = zero
                t1 = zero
                p0 = zero
                p1 = zero
                for gi in range(RW // 8):
                    s0 = zero
                    s1 = zero
                    for s in range(8 * gi, 8 * gi + 8):
                        s0 = s0 + rv[s, pl.ds(0, 16)]
                        s1 = s1 + rv[s, pl.ds(16, 16)]
                    t0 = t0 + s0
                    t1 = t1 + s1
                    pm = jnp.where(8 * gi < sp, 1.0, 0.0)
                    p0 = p0 + pm * s0
                    p1 = p1 + pm * s1
                ct = izero
                cp = izero
                spv = jnp.full((16,), sp, jnp.int32)
                for o in range(0, RW, 16):
                    v = idx_v[r, pl.ds(o, 16)]
                    z = v == 0
                    ct = ct + jnp.where(z, ione, izero)
                    cp = cp + jnp.where(
                        jnp.logical_and(z, (o + lane) < spv), ione, izero
                    )

                @pl.when(more)
                def _(r=r, b=b):
                    fire(r + NBUF, b)

                fin0 = acc0 + p0
                fin1 = acc1 + p1
                finc = cnt + cp
                has_b = sp <= RW

                @pl.when(has_b)
                def _(fin0=fin0, fin1=fin1, finc=finc, j=j):
                    nz = _hsum16(finc, hs_v)
                    nzf = nz.astype(jnp.float32)
                    cntf = (HIST_LEN - nz).astype(jnp.float32)
                    out_v[j, pl.ds(0, 16)] = (fin0 - nzf * w0a) / cntf
                    out_v[j, pl.ds(16, 16)] = (fin1 - nzf * w0b) / cntf

                hbf = jnp.where(has_b, 1.0, 0.0)
                hbi = jnp.where(has_b, 1, 0)
                acc0 = acc0 + t0 - hbf * fin0
                acc1 = acc1 + t1 - hbf * fin1
                cnt = cnt + ct - hbi * finc
                j = j + hbi
                bpos = bpos + HIST_LEN * hbi
            return (acc0, acc1, cnt, j, bpos)

        j0 = p * PHASE_BROWS
        lax.fori_loop(
            0, NGROUP, group, (zero, zero, izero, j0, jnp.int32(HIST_LEN))
        )
        return pcarry

    lax.fori_loop(0, NPHASE, phase, 0)

    pltpu.sync_copy(out_v, out_hbm.at[pl.ds(wid * ROWS_PER_W, ROWS_PER_W)])


@functools.partial(jax.jit, donate_argnums=())
def kernel(X, W):
    xf = X.astype(jnp.int32).reshape(XROWS, RW)
    mesh = plsc.VectorSubcoreMesh(core_axis_name="c", subcore_axis_name="s")
    k = pl.kernel(
        _body,
        mesh=mesh,
        out_type=jax.ShapeDtypeStruct((BATCH, EMB_DIM), jnp.float32),
        scratch_types=[
            pltpu.VMEM((PHASE_XR, RW), jnp.int32),
            [pltpu.VMEM((RW, EMB_DIM), jnp.float32) for _ in range(NBUF)],
            pltpu.VMEM((ROWS_PER_W, EMB_DIM), jnp.float32),
            pltpu.VMEM((8, EMB_DIM), jnp.float32),
            pltpu.VMEM((16,), jnp.int32),
            [pltpu.SemaphoreType.DMA for _ in range(NBUF)],
        ],
        compiler_params=pltpu.CompilerParams(
            needs_layout_passes=False, use_tc_tiling_on_sc=False
        ),
    )
    return k(xf, W)


# speedup vs baseline: 8.4835x; 1.0452x over previous
import functools

import jax
import jax.numpy as jnp
from jax import lax
from jax.experimental import pallas as pl
from jax.experimental.pallas import tpu as pltpu
from jax.experimental.pallas import tpu_sc as plsc

VOC_SIZE = 1000000
EMB_DIM = 32
BATCH = 16384
HIST_LEN = 200
WIN_A = 104
WIN_B = HIST_LEN - WIN_A

_info = plsc.get_sparse_core_info()
NC = _info.num_cores
NS = _info.num_subcores
NW = NC * NS
ROWS_PER_W = BATCH // NW
PHASE_ROWS = 128
NPHASE = ROWS_PER_W // PHASE_ROWS
PHASE_IDX = PHASE_ROWS * HIST_LEN
NBUF = 4
NGROUP = PHASE_ROWS // NBUF


def _hsum16(vec, scratch_ref):
    lane = lax.iota(jnp.int32, 16)
    for sh in (8, 4, 2, 1):
        scratch_ref[...] = vec
        vec = vec + plsc.load_gather(scratch_ref, [lane ^ sh])
    return vec


def _body(x_hbm, w_hbm, out_hbm, idx_v, bufs, out_v, w0_v, hs_v, sems):
    wid = lax.axis_index("s") * NC + lax.axis_index("c")

    pltpu.sync_copy(w_hbm.at[pl.ds(0, 8)], w0_v)
    w0a = w0_v[0, pl.ds(0, 16)]
    w0b = w0_v[0, pl.ds(16, 16)]

    zero = jnp.zeros((16,), jnp.float32)
    izero = jnp.zeros((16,), jnp.int32)
    ione = jnp.ones((16,), jnp.int32)
    lane = lax.iota(jnp.int32, 16)

    def fire(j, b):
        jb = j * HIST_LEN
        pltpu.async_copy(
            w_hbm.at[idx_v.at[pl.ds(jb, WIN_A)]],
            bufs[b].at[pl.ds(0, WIN_A)],
            sems[b],
        )
        pltpu.async_copy(
            w_hbm.at[idx_v.at[pl.ds(jb + WIN_A, WIN_B)]],
            bufs[b].at[pl.ds(WIN_A, WIN_B)],
            sems[b],
        )

    def drain(j, b):
        jb = j * HIST_LEN
        pltpu.make_async_copy(
            w_hbm.at[idx_v.at[pl.ds(jb, WIN_A)]],
            bufs[b].at[pl.ds(0, WIN_A)],
            sems[b],
        ).wait()
        pltpu.make_async_copy(
            w_hbm.at[idx_v.at[pl.ds(jb + WIN_A, WIN_B)]],
            bufs[b].at[pl.ds(WIN_A, WIN_B)],
            sems[b],
        ).wait()

    def phase(p, pcarry):
        pltpu.sync_copy(
            x_hbm.at[pl.ds((wid * NPHASE + p) * PHASE_IDX, PHASE_IDX)], idx_v
        )
        for b in range(NBUF):
            fire(b, b)

        def group(g, carry):
            j0 = g * NBUF
            more = g < NGROUP - 1
            for b in range(NBUF):
                j = j0 + b
                drain(j, b)
                rv = bufs[b]
                t0 = zero
                t1 = zero
                for gi in range(HIST_LEN // 8):
                    s0 = zero
                    s1 = zero
                    for s in range(8 * gi, 8 * gi + 8):
                        s0 = s0 + rv[s, pl.ds(0, 16)]
                        s1 = s1 + rv[s, pl.ds(16, 16)]
                    t0 = t0 + s0
                    t1 = t1 + s1
                cnt = izero
                jb = j * HIST_LEN
                for o in range(0, HIST_LEN - 16, 16):
                    v = idx_v[pl.ds(jb + o, 16)]
                    cnt = cnt + jnp.where(v == 0, ione, izero)
                v = idx_v[pl.ds(jb + HIST_LEN - 16, 16)]
                cnt = cnt + jnp.where(
                    jnp.logical_and(v == 0, lane >= 8), ione, izero
                )

                @pl.when(more)
                def _(j=j, b=b):
                    fire(j + NBUF, b)

                nz = _hsum16(cnt, hs_v)
                nzf = nz.astype(jnp.float32)
                cntf = (HIST_LEN - nz).astype(jnp.float32)
                orow = p * PHASE_ROWS + j
                out_v[orow, pl.ds(0, 16)] = (t0 - nzf * w0a) / cntf
                out_v[orow, pl.ds(16, 16)] = (t1 - nzf * w0b) / cntf
            return carry

        lax.fori_loop(0, NGROUP, group, 0)
        return pcarry

    lax.fori_loop(0, NPHASE, phase, 0)

    pltpu.sync_copy(out_v, out_hbm.at[pl.ds(wid * ROWS_PER_W, ROWS_PER_W)])


@functools.partial(jax.jit, donate_argnums=())
def kernel(X, W):
    xf = X.astype(jnp.int32).reshape(BATCH * HIST_LEN)
    mesh = plsc.VectorSubcoreMesh(core_axis_name="c", subcore_axis_name="s")
    k = pl.kernel(
        _body,
        mesh=mesh,
        out_type=jax.ShapeDtypeStruct((BATCH, EMB_DIM), jnp.float32),
        scratch_types=[
            pltpu.VMEM((PHASE_IDX,), jnp.int32),
            [
                pltpu.VMEM((HIST_LEN, EMB_DIM), jnp.float32)
                for _ in range(NBUF)
            ],
            pltpu.VMEM((ROWS_PER_W, EMB_DIM), jnp.float32),
            pltpu.VMEM((8, EMB_DIM), jnp.float32),
            pltpu.VMEM((16,), jnp.int32),
            [pltpu.SemaphoreType.DMA for _ in range(NBUF)],
        ],
        compiler_params=pltpu.CompilerParams(
            needs_layout_passes=False, use_tc_tiling_on_sc=False
        ),
    )
    return k(xf, W)
